# baseline (speedup 1.0000x reference)
import functools

import jax
import jax.numpy as jnp
from jax import lax
from jax.experimental import pallas as pl
from jax.experimental.pallas import tpu as pltpu
from jax.experimental.pallas import tpu_sc as plsc

N = 65536
PL = 16
NT = 16
CHUNK = N // NT
NV = CHUNK // 16
NJ = CHUNK // 128
NB = 1024
VROW = 48
OUT_CHUNK = N // 32
WIN = 512

_PARAMS = pltpu.CompilerParams(
    needs_layout_passes=False, use_tc_tiling_on_sc=False)


def _iota16():
  return lax.iota(jnp.int32, 16)


def _build_sort_kernel():
  mesh = plsc.VectorSubcoreMesh(
      core_axis_name="c", subcore_axis_name="s", num_cores=1)

  @functools.partial(
      pl.kernel,
      mesh=mesh,
      compiler_params=_PARAMS,
      out_type=jax.ShapeDtypeStruct((N,), jnp.int32),
      scratch_types=[
          pltpu.VMEM((CHUNK,), jnp.int32),
          pltpu.VMEM((CHUNK,), jnp.int32),
          pltpu.VMEM((CHUNK,), jnp.int32),
          pltpu.VMEM((NJ, 128), jnp.int32),
          pltpu.VMEM((NB,), jnp.int32),
          pltpu.VMEM((NB,), jnp.int32),
          pltpu.VMEM((NT, NB), jnp.int32),
          pltpu.VMEM((CHUNK,), jnp.int32),
          pltpu.VMEM((CHUNK,), jnp.int32),
          pltpu.VMEM((CHUNK,), jnp.int32),
          pltpu.VMEM((8,), jnp.int32),
          pltpu.VMEM((16,), jnp.int32),
          pltpu.VMEM((16,), jnp.int32),
          pltpu.VMEM((16,), jnp.int32),
          pltpu.VMEM((16,), jnp.int32),
          pltpu.VMEM((NT, 16), jnp.int32),
          pltpu.VMEM_SHARED((N,), jnp.int32),
          pltpu.VMEM_SHARED((N,), jnp.int32),
          pltpu.VMEM_SHARED((NT, NB), jnp.int32),
          pltpu.VMEM_SHARED((NT, 16), jnp.int32),
          pltpu.SemaphoreType.DMA,
      ],
  )
  def sort_kernel(objf_hbm, groups_hbm, idxc, fidx, colv, posb,
                  hist, offs, hall, flags, fbase, cmcol, pidx, gidx,
                  prow16, frow16, t16, tall,
                  idx_a, idx_b, hsp, tsp, sem):
    sid = lax.axis_index("s")
    base = sid * CHUNK
    lanes = _iota16()
    ones = jnp.ones((16,), jnp.int32)
    zeros = jnp.zeros((16,), jnp.int32)

    def init_body(m, _):
      idxc[pl.ds(16 * m, 16)] = base + 16 * m + lanes
      return 0
    lax.fori_loop(0, NV, init_body, 0)
    pltpu.sync_copy(idxc, idx_a.at[pl.ds(base, CHUNK)])
    plsc.subcore_barrier()

    def one_pass(src_sp, dst_sp, col):
      with jax.named_scope("rs_gather"):
        pltpu.sync_copy(src_sp.at[pl.ds(base, CHUNK)], idxc)
        def fi_body(m, _):
          fidx[pl.ds(16 * m, 16)] = idxc[pl.ds(16 * m, 16)] * PL + col
          return 0
        lax.fori_loop(0, NV, fi_body, 0)
        for j in range(NJ):
          pltpu.async_copy(
              objf_hbm.at[fidx.at[pl.ds(128 * j, 128)]],
              colv.at[pl.ds(128 * j, 128)], sem)
        pltpu.make_async_copy(objf_hbm.at[pl.ds(0, CHUNK)], colv, sem).wait()
      with jax.named_scope("rs_hist"):
        def hz_body(m, _):
          hist[pl.ds(16 * m, 16)] = zeros
          return 0
        lax.fori_loop(0, NB // 16, hz_body, 0)
        def dig_body(m, _):
          plsc.addupdate_scatter(hist, [colv[pl.ds(16 * m, 16)]], ones)
          return 0
        lax.fori_loop(0, NV, dig_body, 0)
      with jax.named_scope("rs_xchg"):
        pltpu.sync_copy(hist, hsp.at[sid])
        plsc.subcore_barrier()
        pltpu.sync_copy(hsp, hall)
      with jax.named_scope("rs_scan"):
        def scan_body(k, carry):
          tot = zeros
          part = zeros
          for t in range(NT):
            h = hall[t, pl.ds(16 * k, 16)]
            tot = tot + h
            part = part + h * jnp.where(jnp.int32(t) < sid, 1, 0)
          incl = plsc.cumsum(tot)
          offs[pl.ds(16 * k, 16)] = carry + (incl - tot) + part
          return carry + jnp.sum(tot)
        lax.fori_loop(0, NB // 16, scan_body, jnp.int32(0))
      with jax.named_scope("rs_perm"):
        def perm_body(m, _):
          d = colv[pl.ds(16 * m, 16)]
          b = plsc.load_gather(offs, [d])
          cnt, _ = plsc.scan_count(d)
          posb[m // 8, pl.ds((m % 8) * 16, 16)] = b + cnt - 1
          plsc.addupdate_scatter(offs, [d], ones)
          return 0
        lax.fori_loop(0, NV, perm_body, 0)
      with jax.named_scope("rs_scat"):
        for j in range(NJ):
          pltpu.async_copy(
              idxc.at[pl.ds(128 * j, 128)], dst_sp.at[posb.at[j]], sem)
        pltpu.make_async_copy(groups_hbm.at[pl.ds(0, CHUNK)], fidx, sem).wait()
        plsc.subcore_barrier()

    def two_passes(k, _):
      one_pass(idx_a, idx_b, 15 - 2 * k)
      one_pass(idx_b, idx_a, 14 - 2 * k)
      return 0
    lax.fori_loop(0, 8, two_passes, 0)

    pltpu.sync_copy(idx_a.at[pl.ds(base, CHUNK)], idxc)
    def fb_body(m, _):
      fbase[pl.ds(16 * m, 16)] = idxc[pl.ds(16 * m, 16)] * PL
      return 0
    lax.fori_loop(0, NV, fb_body, 0)
    pb = pl.multiple_of(jnp.maximum(base - 8, 0), 8)
    pltpu.sync_copy(idx_a.at[pl.ds(pb, 8)], pidx)
    pv = plsc.load_gather(pidx, [jnp.full((16,), 7, jnp.int32)])
    gidx[...] = pv * PL + lanes
    pltpu.async_copy(objf_hbm.at[gidx], prow16, sem).wait()
    fv = plsc.load_gather(idxc, [zeros])
    gidx[...] = fv * PL + lanes
    pltpu.async_copy(objf_hbm.at[gidx], frow16, sem).wait()

    for jcol in range(PL):
      def fc_body(m, _):
        fidx[pl.ds(16 * m, 16)] = fbase[pl.ds(16 * m, 16)] + jcol
        return 0
      lax.fori_loop(0, NV, fc_body, 0)
      for j in range(NJ):
        pltpu.async_copy(
            objf_hbm.at[fidx.at[pl.ds(128 * j, 128)]],
            cmcol.at[pl.ds(128 * j, 128)], sem)
      pltpu.make_async_copy(objf_hbm.at[pl.ds(0, CHUNK)], cmcol, sem).wait()
      def cmp_body(m, _):
        p = 16 * m + lanes
        pp = jnp.maximum(p - 1, 0)
        cur = cmcol[pl.ds(16 * m, 16)]
        prv = plsc.load_gather(cmcol, [pp])
        neq = jnp.where(cur != prv, 1, 0)
        if jcol == 0:
          flags[pl.ds(16 * m, 16)] = neq
        else:
          flags[pl.ds(16 * m, 16)] = flags[pl.ds(16 * m, 16)] | neq
        return 0
      lax.fori_loop(0, NV, cmp_body, 0)
    df = jnp.sum(jnp.where(frow16[...] != prow16[...], 1, 0))
    f0 = jnp.where(sid == 0, jnp.int32(0), jnp.minimum(df, 1))
    v0 = flags[pl.ds(0, 16)]
    flags[pl.ds(0, 16)] = jnp.where(lanes == 0, f0, v0)

    def sum_body(m, carry):
      f = flags[pl.ds(16 * m, 16)]
      colv[pl.ds(16 * m, 16)] = plsc.cumsum(f) + carry
      return carry + jnp.sum(f)
    tot = lax.fori_loop(0, NV, sum_body, jnp.int32(0))
    t16[...] = zeros + tot
    pltpu.sync_copy(t16, tsp.at[sid])
    plsc.subcore_barrier()
    pltpu.sync_copy(tsp, tall)
    rbase = zeros
    for t in range(NT):
      rbase = rbase + tall[t, :] * jnp.where(jnp.int32(t) < sid, 1, 0)
    def add_body(m, _):
      colv[pl.ds(16 * m, 16)] = colv[pl.ds(16 * m, 16)] + rbase
      posb[m // 8, pl.ds((m % 8) * 16, 16)] = idxc[pl.ds(16 * m, 16)]
      return 0
    lax.fori_loop(0, NV, add_body, 0)
    for j in range(NJ):
      pltpu.async_copy(
          colv.at[pl.ds(128 * j, 128)], groups_hbm.at[posb.at[j]], sem)
    pltpu.make_async_copy(groups_hbm.at[pl.ds(0, CHUNK)], fidx, sem).wait()

  return sort_kernel


def _build_vertex_kernel():
  mesh = plsc.VectorSubcoreMesh(
      core_axis_name="c", subcore_axis_name="s", num_cores=2)

  @functools.partial(
      pl.kernel,
      mesh=mesh,
      compiler_params=_PARAMS,
      out_type=jax.ShapeDtypeStruct((N, VROW), jnp.float32),
      scratch_types=[
          pltpu.VMEM((CHUNK,), jnp.int32),
          pltpu.VMEM((CHUNK,), jnp.int32),
          pltpu.VMEM((NJ, 128), jnp.int32),
          pltpu.VMEM((WIN,), jnp.int32),
          pltpu.VMEM((WIN, VROW), jnp.float32),
          pltpu.VMEM((16,), jnp.int32),
          pltpu.VMEM((NT, 16), jnp.int32),
          pltpu.VMEM_SHARED((N + 128,), jnp.int32),
          pltpu.VMEM_SHARED((NT, 16), jnp.int32),
          pltpu.SemaphoreType.DMA,
      ],
  )
  def vertex_kernel(mask_hbm, vert_hbm, out_hbm, mch, posn, posb, widx,
                    wrows, t16, tall, isp, tsp, sem):
    cid = lax.axis_index("c")
    sid = lax.axis_index("s")
    base = sid * CHUNK
    lanes = _iota16()
    zeros = jnp.zeros((16,), jnp.int32)

    def wz_body(m, _):
      widx[pl.ds(16 * m, 16)] = zeros
      return 0
    lax.fori_loop(0, WIN // 16, wz_body, 0)
    for k in range(CHUNK // WIN):
      pltpu.sync_copy(widx, isp.at[pl.ds(base + k * WIN, WIN)])
    @pl.when(sid == 0)
    def _():
      pltpu.sync_copy(widx.at[pl.ds(0, 128)], isp.at[pl.ds(N, 128)])
    pltpu.sync_copy(mask_hbm.at[pl.ds(base, CHUNK)], mch)
    def cs_body(m, carry):
      v = mch[pl.ds(16 * m, 16)]
      posn[pl.ds(16 * m, 16)] = (plsc.cumsum(v) - v) + carry
      return carry + jnp.sum(v)
    tot = lax.fori_loop(0, NV, cs_body, jnp.int32(0))
    t16[...] = zeros + tot
    pltpu.sync_copy(t16, tsp.at[sid])
    plsc.subcore_barrier()
    pltpu.sync_copy(tsp, tall)
    cbase = zeros
    for t in range(NT):
      cbase = cbase + tall[t, :] * jnp.where(jnp.int32(t) < sid, 1, 0)
    def ps_body(m, _):
      v = mch[pl.ds(16 * m, 16)]
      p = posn[pl.ds(16 * m, 16)] + cbase
      dump = jnp.full((16,), N, jnp.int32) + lanes
      posn[pl.ds(16 * m, 16)] = jnp.where(v > 0, p, dump)
      mch[pl.ds(16 * m, 16)] = base + 16 * m + lanes
      posb[m // 8, pl.ds((m % 8) * 16, 16)] = jnp.where(v > 0, p, dump)
      return 0
    lax.fori_loop(0, NV, ps_body, 0)
    for j in range(NJ):
      pltpu.async_copy(mch.at[pl.ds(128 * j, 128)], isp.at[posb.at[j]], sem)
    pltpu.make_async_copy(mask_hbm.at[pl.ds(0, CHUNK)], posn, sem).wait()
    plsc.subcore_barrier()
    w = cid * NT + sid
    for win in range(OUT_CHUNK // WIN):
      start = w * OUT_CHUNK + win * WIN
      pltpu.sync_copy(isp.at[pl.ds(start, WIN)], widx)
      for j in range(WIN // 128):
        pltpu.async_copy(
            vert_hbm.at[widx.at[pl.ds(128 * j, 128)]],
            wrows.at[pl.ds(128 * j, 128)], sem)
      pltpu.make_async_copy(vert_hbm.at[pl.ds(0, WIN)], wrows, sem).wait()
      pltpu.sync_copy(wrows, out_hbm.at[pl.ds(start, WIN)])

  return vertex_kernel


_sort_call = _build_sort_kernel()
_vertex_call = _build_vertex_kernel()


def kernel(vertices, objects, mask):
  path_len = vertices.shape[-2]
  objf = objects.reshape(-1).astype(jnp.int32)
  groups = _sort_call(objf)
  v2d = vertices.reshape(-1, path_len * 3).astype(jnp.float32)
  if mask is not None:
    m32 = mask.reshape(-1).astype(jnp.int32)
    mv = _vertex_call(m32, v2d)
  else:
    mv = _vertex_call(jnp.ones((v2d.shape[0],), jnp.int32), v2d)
  masked_vertices = mv.reshape(-1, path_len, 3)
  groups = groups.reshape(objects.shape[:-1])
  return masked_vertices, groups


# speedup vs baseline: 1.2173x; 1.2173x over previous
import functools

import jax
import jax.numpy as jnp
from jax import lax
from jax.experimental import pallas as pl
from jax.experimental.pallas import tpu as pltpu
from jax.experimental.pallas import tpu_sc as plsc

N = 65536
PL = 16
NT = 16
CHUNK = N // NT
NV = CHUNK // 16
NJ = CHUNK // 128
NB = 1024
VROW = 48
OUT_CHUNK = N // 32
WIN = 512

_PARAMS = pltpu.CompilerParams(
    needs_layout_passes=False, use_tc_tiling_on_sc=False)


def _iota16():
  return lax.iota(jnp.int32, 16)


def _build_sort_kernel():
  mesh = plsc.VectorSubcoreMesh(
      core_axis_name="c", subcore_axis_name="s", num_cores=1)

  @functools.partial(
      pl.kernel,
      mesh=mesh,
      compiler_params=_PARAMS,
      out_type=jax.ShapeDtypeStruct((N,), jnp.int32),
      scratch_types=[
          pltpu.VMEM((CHUNK,), jnp.int32),
          pltpu.VMEM((CHUNK,), jnp.int32),
          pltpu.VMEM((CHUNK,), jnp.int32),
          pltpu.VMEM((CHUNK,), jnp.int32),
          pltpu.VMEM((NJ, 128), jnp.int32),
          pltpu.VMEM((NB,), jnp.int32),
          pltpu.VMEM((NB,), jnp.int32),
          pltpu.VMEM((NT, NB), jnp.int32),
          pltpu.VMEM((CHUNK,), jnp.int32),
          pltpu.VMEM((CHUNK, PL), jnp.int32),
          pltpu.VMEM((8,), jnp.int32),
          pltpu.VMEM((8, PL), jnp.int32),
          pltpu.VMEM((16,), jnp.int32),
          pltpu.VMEM((NT, 16), jnp.int32),
          pltpu.VMEM_SHARED((2 * N,), jnp.int32),
          pltpu.VMEM_SHARED((2 * N,), jnp.int32),
          pltpu.VMEM_SHARED((NT, NB), jnp.int32),
          pltpu.VMEM_SHARED((NT, 16), jnp.int32),
          pltpu.SemaphoreType.DMA,
      ],
  )
  def sort_kernel(obj2d_hbm, groups_hbm, idxc, fidx, colv, pkdc, posb,
                  hist, offs, hall, flags, rows, pidx, prow, t16, tall,
                  idxsp, pkdsp, hsp, tsp, sem):
    sid = lax.axis_index("s")
    base = sid * CHUNK
    lanes = _iota16()
    ones = jnp.ones((16,), jnp.int32)
    zeros = jnp.zeros((16,), jnp.int32)

    def init_body(m, _):
      idxc[pl.ds(16 * m, 16)] = base + 16 * m + lanes
      return 0
    lax.fori_loop(0, NV, init_body, 0)
    pltpu.sync_copy(idxc, idxsp.at[pl.ds(base, CHUNK)])
    plsc.subcore_barrier()

    def pass_body(p, _):
      col = 15 - p
      r = lax.rem(p, 3)
      src = pl.multiple_of(lax.rem(p, 2) * N + base, 8)
      dst_off = (1 - lax.rem(p, 2)) * N
      with jax.named_scope("rs_gather"):
        pltpu.sync_copy(idxsp.at[pl.ds(src, CHUNK)], idxc)
        @pl.when(r == 0)
        def _():
          for j in range(NJ):
            pltpu.async_copy(
                obj2d_hbm.at[idxc.at[pl.ds(128 * j, 128)]],
                rows.at[pl.ds(128 * j, 128)], sem)
          pltpu.make_async_copy(
              obj2d_hbm.at[pl.ds(0, CHUNK)], rows, sem).wait()
          c0 = zeros + col
          c1 = jnp.maximum(c0 - 1, 0)
          c2 = jnp.maximum(c0 - 2, 0)
          def ex_body(m, _):
            pos = 16 * m + lanes
            colv[pl.ds(16 * m, 16)] = plsc.load_gather(rows, [pos, c0])
            d1 = plsc.load_gather(rows, [pos, c1])
            d2 = plsc.load_gather(rows, [pos, c2])
            pkdc[pl.ds(16 * m, 16)] = d1 * 1024 + d2
            return 0
          lax.fori_loop(0, NV, ex_body, 0)
        @pl.when(r != 0)
        def _():
          pltpu.sync_copy(pkdsp.at[pl.ds(src, CHUNK)], pkdc)
          @pl.when(r == 1)
          def _():
            def s1_body(m, _):
              v = pkdc[pl.ds(16 * m, 16)]
              colv[pl.ds(16 * m, 16)] = lax.shift_right_logical(v, 10)
              pkdc[pl.ds(16 * m, 16)] = jnp.bitwise_and(v, 1023)
              return 0
            lax.fori_loop(0, NV, s1_body, 0)
          @pl.when(r == 2)
          def _():
            def s2_body(m, _):
              colv[pl.ds(16 * m, 16)] = pkdc[pl.ds(16 * m, 16)]
              return 0
            lax.fori_loop(0, NV, s2_body, 0)
      with jax.named_scope("rs_hist"):
        def hz_body(m, _):
          hist[pl.ds(16 * m, 16)] = zeros
          return 0
        lax.fori_loop(0, NB // 16, hz_body, 0)
        def dig_body(m, _):
          plsc.addupdate_scatter(hist, [colv[pl.ds(16 * m, 16)]], ones)
          return 0
        lax.fori_loop(0, NV, dig_body, 0)
      with jax.named_scope("rs_xchg"):
        pltpu.sync_copy(hist, hsp.at[sid])
        plsc.subcore_barrier()
        pltpu.sync_copy(hsp, hall)
      with jax.named_scope("rs_scan"):
        def scan_body(k, carry):
          tot = zeros
          part = zeros
          for t in range(NT):
            h = hall[t, pl.ds(16 * k, 16)]
            tot = tot + h
            part = part + h * jnp.where(jnp.int32(t) < sid, 1, 0)
          incl = plsc.cumsum(tot)
          offs[pl.ds(16 * k, 16)] = carry + (incl - tot) + part
          return carry + jnp.sum(tot)
        lax.fori_loop(0, NB // 16, scan_body, jnp.int32(0))
      with jax.named_scope("rs_perm"):
        dvec = zeros + dst_off
        def perm_body(m, _):
          d = colv[pl.ds(16 * m, 16)]
          b = plsc.load_gather(offs, [d])
          cnt, _ = plsc.scan_count(d)
          posb[m // 8, pl.ds((m % 8) * 16, 16)] = b + cnt - 1 + dvec
          plsc.addupdate_scatter(offs, [d], ones)
          return 0
        lax.fori_loop(0, NV, perm_body, 0)
      with jax.named_scope("rs_scat"):
        for j in range(NJ):
          pltpu.async_copy(
              idxc.at[pl.ds(128 * j, 128)], idxsp.at[posb.at[j]], sem)
        pltpu.make_async_copy(
            groups_hbm.at[pl.ds(0, CHUNK)], fidx, sem).wait()
        @pl.when(jnp.logical_and(r != 2, p != 15))
        def _():
          for j in range(NJ):
            pltpu.async_copy(
                pkdc.at[pl.ds(128 * j, 128)], pkdsp.at[posb.at[j]], sem)
          pltpu.make_async_copy(
              groups_hbm.at[pl.ds(0, CHUNK)], fidx, sem).wait()
        plsc.subcore_barrier()
      return 0
    lax.fori_loop(0, 16, pass_body, 0)

    pltpu.sync_copy(idxsp.at[pl.ds(base, CHUNK)], idxc)
    for j in range(NJ):
      pltpu.async_copy(
          obj2d_hbm.at[idxc.at[pl.ds(128 * j, 128)]],
          rows.at[pl.ds(128 * j, 128)], sem)
    pltpu.make_async_copy(obj2d_hbm.at[pl.ds(0, CHUNK)], rows, sem).wait()
    pb = pl.multiple_of(jnp.maximum(base - 8, 0), 8)
    pltpu.sync_copy(idxsp.at[pl.ds(pb, 8)], pidx)
    pltpu.async_copy(obj2d_hbm.at[pidx], prow, sem).wait()

    def cmp_body(m, _):
      p = 16 * m + lanes
      pp = jnp.maximum(p - 1, 0)
      acc = zeros
      for jcol in range(PL):
        cj = jnp.full((16,), jcol, jnp.int32)
        cur = plsc.load_gather(rows, [p, cj])
        prv = plsc.load_gather(rows, [pp, cj])
        acc = acc | jnp.where(cur != prv, 1, 0)
      flags[pl.ds(16 * m, 16)] = acc
      return 0
    lax.fori_loop(0, NV, cmp_body, 0)
    first = rows[0, :]
    prev = prow[7, :]
    df = jnp.sum(jnp.where(first != prev, 1, 0))
    f0 = jnp.where(sid == 0, jnp.int32(0), jnp.minimum(df, 1))
    v0 = flags[pl.ds(0, 16)]
    flags[pl.ds(0, 16)] = jnp.where(lanes == 0, f0, v0)

    def sum_body(m, carry):
      f = flags[pl.ds(16 * m, 16)]
      colv[pl.ds(16 * m, 16)] = plsc.cumsum(f) + carry
      return carry + jnp.sum(f)
    tot = lax.fori_loop(0, NV, sum_body, jnp.int32(0))
    t16[...] = zeros + tot
    pltpu.sync_copy(t16, tsp.at[sid])
    plsc.subcore_barrier()
    pltpu.sync_copy(tsp, tall)
    rbase = zeros
    for t in range(NT):
      rbase = rbase + tall[t, :] * jnp.where(jnp.int32(t) < sid, 1, 0)
    def add_body(m, _):
      colv[pl.ds(16 * m, 16)] = colv[pl.ds(16 * m, 16)] + rbase
      posb[m // 8, pl.ds((m % 8) * 16, 16)] = idxc[pl.ds(16 * m, 16)]
      return 0
    lax.fori_loop(0, NV, add_body, 0)
    for j in range(NJ):
      pltpu.async_copy(
          colv.at[pl.ds(128 * j, 128)], groups_hbm.at[posb.at[j]], sem)
    pltpu.make_async_copy(groups_hbm.at[pl.ds(0, CHUNK)], fidx, sem).wait()

  return sort_kernel


def _build_vertex_kernel():
  mesh = plsc.VectorSubcoreMesh(
      core_axis_name="c", subcore_axis_name="s", num_cores=2)

  @functools.partial(
      pl.kernel,
      mesh=mesh,
      compiler_params=_PARAMS,
      out_type=jax.ShapeDtypeStruct((N, VROW), jnp.float32),
      scratch_types=[
          pltpu.VMEM((CHUNK,), jnp.int32),
          pltpu.VMEM((CHUNK,), jnp.int32),
          pltpu.VMEM((NJ, 128), jnp.int32),
          pltpu.VMEM((WIN,), jnp.int32),
          pltpu.VMEM((WIN, VROW), jnp.float32),
          pltpu.VMEM((16,), jnp.int32),
          pltpu.VMEM((NT, 16), jnp.int32),
          pltpu.VMEM_SHARED((N + 128,), jnp.int32),
          pltpu.VMEM_SHARED((NT, 16), jnp.int32),
          pltpu.SemaphoreType.DMA,
      ],
  )
  def vertex_kernel(mask_hbm, vert_hbm, out_hbm, mch, posn, posb, widx,
                    wrows, t16, tall, isp, tsp, sem):
    cid = lax.axis_index("c")
    sid = lax.axis_index("s")
    base = sid * CHUNK
    lanes = _iota16()
    zeros = jnp.zeros((16,), jnp.int32)

    def wz_body(m, _):
      widx[pl.ds(16 * m, 16)] = zeros
      return 0
    lax.fori_loop(0, WIN // 16, wz_body, 0)
    for k in range(CHUNK // WIN):
      pltpu.sync_copy(widx, isp.at[pl.ds(base + k * WIN, WIN)])
    @pl.when(sid == 0)
    def _():
      pltpu.sync_copy(widx.at[pl.ds(0, 128)], isp.at[pl.ds(N, 128)])
    pltpu.sync_copy(mask_hbm.at[pl.ds(base, CHUNK)], mch)
    def cs_body(m, carry):
      v = mch[pl.ds(16 * m, 16)]
      posn[pl.ds(16 * m, 16)] = (plsc.cumsum(v) - v) + carry
      return carry + jnp.sum(v)
    tot = lax.fori_loop(0, NV, cs_body, jnp.int32(0))
    t16[...] = zeros + tot
    pltpu.sync_copy(t16, tsp.at[sid])
    plsc.subcore_barrier()
    pltpu.sync_copy(tsp, tall)
    cbase = zeros
    for t in range(NT):
      cbase = cbase + tall[t, :] * jnp.where(jnp.int32(t) < sid, 1, 0)
    def ps_body(m, _):
      v = mch[pl.ds(16 * m, 16)]
      p = posn[pl.ds(16 * m, 16)] + cbase
      dump = jnp.full((16,), N, jnp.int32) + lanes
      pd = jnp.where(v > 0, p, dump)
      posn[pl.ds(16 * m, 16)] = pd
      mch[pl.ds(16 * m, 16)] = base + 16 * m + lanes
      posb[m // 8, pl.ds((m % 8) * 16, 16)] = pd
      return 0
    lax.fori_loop(0, NV, ps_body, 0)
    for j in range(NJ):
      pltpu.async_copy(mch.at[pl.ds(128 * j, 128)], isp.at[posb.at[j]], sem)
    pltpu.make_async_copy(mask_hbm.at[pl.ds(0, CHUNK)], posn, sem).wait()
    plsc.subcore_barrier()
    w = cid * NT + sid
    for win in range(OUT_CHUNK // WIN):
      start = w * OUT_CHUNK + win * WIN
      pltpu.sync_copy(isp.at[pl.ds(start, WIN)], widx)
      for j in range(WIN // 128):
        pltpu.async_copy(
            vert_hbm.at[widx.at[pl.ds(128 * j, 128)]],
            wrows.at[pl.ds(128 * j, 128)], sem)
      pltpu.make_async_copy(vert_hbm.at[pl.ds(0, WIN)], wrows, sem).wait()
      pltpu.sync_copy(wrows, out_hbm.at[pl.ds(start, WIN)])

  return vertex_kernel


_sort_call = _build_sort_kernel()
_vertex_call = _build_vertex_kernel()


def kernel(vertices, objects, mask):
  path_len = vertices.shape[-2]
  obj2d = objects.reshape(-1, objects.shape[-1]).astype(jnp.int32)
  groups = _sort_call(obj2d)
  v2d = vertices.reshape(-1, path_len * 3).astype(jnp.float32)
  if mask is not None:
    m32 = mask.reshape(-1).astype(jnp.int32)
    mv = _vertex_call(m32, v2d)
  else:
    mv = _vertex_call(jnp.ones((v2d.shape[0],), jnp.int32), v2d)
  masked_vertices = mv.reshape(-1, path_len, 3)
  groups = groups.reshape(objects.shape[:-1])
  return masked_vertices, groups


# speedup vs baseline: 1.2293x; 1.0098x over previous
import functools

import jax
import jax.numpy as jnp
from jax import lax
from jax.experimental import pallas as pl
from jax.experimental.pallas import tpu as pltpu
from jax.experimental.pallas import tpu_sc as plsc

N = 65536
PL = 16
NT = 16
CHUNK = N // NT
NV = CHUNK // 16
NJ = CHUNK // 128
NB = 1024
VROW = 48
OUT_CHUNK = N // 32
WIN = 512

_PARAMS = pltpu.CompilerParams(
    needs_layout_passes=False, use_tc_tiling_on_sc=False)


def _iota16():
  return lax.iota(jnp.int32, 16)


def _build_sort_kernel():
  mesh = plsc.VectorSubcoreMesh(
      core_axis_name="c", subcore_axis_name="s", num_cores=1)

  @functools.partial(
      pl.kernel,
      mesh=mesh,
      compiler_params=_PARAMS,
      out_type=jax.ShapeDtypeStruct((N,), jnp.int32),
      scratch_types=[
          pltpu.VMEM((CHUNK,), jnp.int32),
          pltpu.VMEM((CHUNK,), jnp.int32),
          pltpu.VMEM((CHUNK,), jnp.int32),
          pltpu.VMEM((CHUNK,), jnp.int32),
          pltpu.VMEM((NJ, 128), jnp.int32),
          pltpu.VMEM((NB,), jnp.int32),
          pltpu.VMEM((NB,), jnp.int32),
          pltpu.VMEM((NT, NB), jnp.int32),
          pltpu.VMEM((CHUNK,), jnp.int32),
          pltpu.VMEM((CHUNK, PL), jnp.int32),
          pltpu.VMEM((8,), jnp.int32),
          pltpu.VMEM((8, PL), jnp.int32),
          pltpu.VMEM((16,), jnp.int32),
          pltpu.VMEM((NT, 16), jnp.int32),
          pltpu.VMEM_SHARED((2 * N,), jnp.int32),
          pltpu.VMEM_SHARED((2 * N,), jnp.int32),
          pltpu.VMEM_SHARED((NT, NB), jnp.int32),
          pltpu.VMEM_SHARED((NT, 16), jnp.int32),
          pltpu.SemaphoreType.DMA,
      ],
  )
  def sort_kernel(obj2d_hbm, groups_hbm, idxc, fidx, colv, pkdc, posb,
                  hist, offs, hall, flags, rows, pidx, prow, t16, tall,
                  idxsp, pkdsp, hsp, tsp, sem):
    sid = lax.axis_index("s")
    base = sid * CHUNK
    lanes = _iota16()
    ones = jnp.ones((16,), jnp.int32)
    zeros = jnp.zeros((16,), jnp.int32)

    def init_body(m, _):
      idxc[pl.ds(16 * m, 16)] = base + 16 * m + lanes
      return 0
    lax.fori_loop(0, NV, init_body, 0)
    pltpu.sync_copy(idxc, idxsp.at[pl.ds(base, CHUNK)])
    plsc.subcore_barrier()

    def pass_body(p, _):
      col = 15 - p
      r = lax.rem(p, 3)
      src = pl.multiple_of(lax.rem(p, 2) * N + base, 8)
      dst_off = (1 - lax.rem(p, 2)) * N
      with jax.named_scope("rs_gather"):
        pltpu.sync_copy(idxsp.at[pl.ds(src, CHUNK)], idxc)
        @pl.when(r == 0)
        def _():
          for j in range(NJ):
            pltpu.async_copy(
                obj2d_hbm.at[idxc.at[pl.ds(128 * j, 128)]],
                rows.at[pl.ds(128 * j, 128)], sem)
          pltpu.make_async_copy(
              obj2d_hbm.at[pl.ds(0, CHUNK)], rows, sem).wait()
          c0 = zeros + col
          c1 = jnp.maximum(c0 - 1, 0)
          c2 = jnp.maximum(c0 - 2, 0)
          def ex_body(m, _):
            for h in range(4):
              mm = 4 * m + h
              pos = 16 * mm + lanes
              colv[pl.ds(16 * mm, 16)] = plsc.load_gather(rows, [pos, c0])
              d1 = plsc.load_gather(rows, [pos, c1])
              d2 = plsc.load_gather(rows, [pos, c2])
              pkdc[pl.ds(16 * mm, 16)] = d1 * 1024 + d2
            return 0
          lax.fori_loop(0, NV // 4, ex_body, 0)
        @pl.when(r != 0)
        def _():
          pltpu.sync_copy(pkdsp.at[pl.ds(src, CHUNK)], pkdc)
          @pl.when(r == 1)
          def _():
            def s1_body(m, _):
              for h in range(4):
                mm = 4 * m + h
                v = pkdc[pl.ds(16 * mm, 16)]
                colv[pl.ds(16 * mm, 16)] = lax.shift_right_logical(v, 10)
                pkdc[pl.ds(16 * mm, 16)] = jnp.bitwise_and(v, 1023)
              return 0
            lax.fori_loop(0, NV // 4, s1_body, 0)
          @pl.when(r == 2)
          def _():
            def s2_body(m, _):
              for h in range(4):
                mm = 4 * m + h
                colv[pl.ds(16 * mm, 16)] = pkdc[pl.ds(16 * mm, 16)]
              return 0
            lax.fori_loop(0, NV // 4, s2_body, 0)
      with jax.named_scope("rs_hist"):
        def hz_body(m, _):
          hist[pl.ds(16 * m, 16)] = zeros
          return 0
        lax.fori_loop(0, NB // 16, hz_body, 0)
        def dig_body(m, _):
          for h in range(4):
            mm = 4 * m + h
            plsc.addupdate_scatter(hist, [colv[pl.ds(16 * mm, 16)]], ones)
          return 0
        lax.fori_loop(0, NV // 4, dig_body, 0)
      with jax.named_scope("rs_xchg"):
        pltpu.sync_copy(hist, hsp.at[sid])
        plsc.subcore_barrier()
        pltpu.sync_copy(hsp, hall)
      with jax.named_scope("rs_scan"):
        def scan_body(k, carry):
          tot = zeros
          part = zeros
          for t in range(NT):
            h = hall[t, pl.ds(16 * k, 16)]
            tot = tot + h
            part = part + h * jnp.where(jnp.int32(t) < sid, 1, 0)
          incl = plsc.cumsum(tot)
          offs[pl.ds(16 * k, 16)] = carry + (incl - tot) + part
          return carry + jnp.sum(tot)
        lax.fori_loop(0, NB // 16, scan_body, jnp.int32(0))
      with jax.named_scope("rs_perm"):
        dvec = zeros + dst_off
        def perm_body(m, _):
          for h in range(2):
            mm = 2 * m + h
            d = colv[pl.ds(16 * mm, 16)]
            b = plsc.load_gather(offs, [d])
            cnt, _ = plsc.scan_count(d)
            posb[mm // 8, pl.ds((mm % 8) * 16, 16)] = b + cnt - 1 + dvec
            plsc.addupdate_scatter(offs, [d], ones)
          return 0
        lax.fori_loop(0, NV // 2, perm_body, 0)
      with jax.named_scope("rs_scat"):
        for j in range(NJ):
          pltpu.async_copy(
              idxc.at[pl.ds(128 * j, 128)], idxsp.at[posb.at[j]], sem)
        pltpu.make_async_copy(
            groups_hbm.at[pl.ds(0, CHUNK)], fidx, sem).wait()
        @pl.when(jnp.logical_and(r != 2, p != 15))
        def _():
          for j in range(NJ):
            pltpu.async_copy(
                pkdc.at[pl.ds(128 * j, 128)], pkdsp.at[posb.at[j]], sem)
          pltpu.make_async_copy(
              groups_hbm.at[pl.ds(0, CHUNK)], fidx, sem).wait()
        plsc.subcore_barrier()
      return 0
    lax.fori_loop(0, 16, pass_body, 0)

    pltpu.sync_copy(idxsp.at[pl.ds(base, CHUNK)], idxc)
    for j in range(NJ):
      pltpu.async_copy(
          obj2d_hbm.at[idxc.at[pl.ds(128 * j, 128)]],
          rows.at[pl.ds(128 * j, 128)], sem)
    pltpu.make_async_copy(obj2d_hbm.at[pl.ds(0, CHUNK)], rows, sem).wait()
    pb = pl.multiple_of(jnp.maximum(base - 8, 0), 8)
    pltpu.sync_copy(idxsp.at[pl.ds(pb, 8)], pidx)
    pltpu.async_copy(obj2d_hbm.at[pidx], prow, sem).wait()

    def cmp_body(m, _):
      p = 16 * m + lanes
      pp = jnp.maximum(p - 1, 0)
      acc = zeros
      for jcol in range(PL):
        cj = jnp.full((16,), jcol, jnp.int32)
        cur = plsc.load_gather(rows, [p, cj])
        prv = plsc.load_gather(rows, [pp, cj])
        acc = acc | jnp.where(cur != prv, 1, 0)
      flags[pl.ds(16 * m, 16)] = acc
      return 0
    lax.fori_loop(0, NV, cmp_body, 0)
    first = rows[0, :]
    prev = prow[7, :]
    df = jnp.sum(jnp.where(first != prev, 1, 0))
    f0 = jnp.where(sid == 0, jnp.int32(0), jnp.minimum(df, 1))
    v0 = flags[pl.ds(0, 16)]
    flags[pl.ds(0, 16)] = jnp.where(lanes == 0, f0, v0)

    def sum_body(m, carry):
      f = flags[pl.ds(16 * m, 16)]
      colv[pl.ds(16 * m, 16)] = plsc.cumsum(f) + carry
      return carry + jnp.sum(f)
    tot = lax.fori_loop(0, NV, sum_body, jnp.int32(0))
    t16[...] = zeros + tot
    pltpu.sync_copy(t16, tsp.at[sid])
    plsc.subcore_barrier()
    pltpu.sync_copy(tsp, tall)
    rbase = zeros
    for t in range(NT):
      rbase = rbase + tall[t, :] * jnp.where(jnp.int32(t) < sid, 1, 0)
    def add_body(m, _):
      colv[pl.ds(16 * m, 16)] = colv[pl.ds(16 * m, 16)] + rbase
      posb[m // 8, pl.ds((m % 8) * 16, 16)] = idxc[pl.ds(16 * m, 16)]
      return 0
    lax.fori_loop(0, NV, add_body, 0)
    for j in range(NJ):
      pltpu.async_copy(
          colv.at[pl.ds(128 * j, 128)], groups_hbm.at[posb.at[j]], sem)
    pltpu.make_async_copy(groups_hbm.at[pl.ds(0, CHUNK)], fidx, sem).wait()

  return sort_kernel


def _build_vertex_kernel():
  mesh = plsc.VectorSubcoreMesh(
      core_axis_name="c", subcore_axis_name="s", num_cores=2)

  @functools.partial(
      pl.kernel,
      mesh=mesh,
      compiler_params=_PARAMS,
      out_type=jax.ShapeDtypeStruct((N, VROW), jnp.float32),
      scratch_types=[
          pltpu.VMEM((CHUNK,), jnp.int32),
          pltpu.VMEM((CHUNK,), jnp.int32),
          pltpu.VMEM((NJ, 128), jnp.int32),
          pltpu.VMEM((WIN,), jnp.int32),
          pltpu.VMEM((WIN, VROW), jnp.float32),
          pltpu.VMEM((16,), jnp.int32),
          pltpu.VMEM((NT, 16), jnp.int32),
          pltpu.VMEM_SHARED((N + 128,), jnp.int32),
          pltpu.VMEM_SHARED((NT, 16), jnp.int32),
          pltpu.SemaphoreType.DMA,
      ],
  )
  def vertex_kernel(mask_hbm, vert_hbm, out_hbm, mch, posn, posb, widx,
                    wrows, t16, tall, isp, tsp, sem):
    cid = lax.axis_index("c")
    sid = lax.axis_index("s")
    base = sid * CHUNK
    lanes = _iota16()
    zeros = jnp.zeros((16,), jnp.int32)

    def wz_body(m, _):
      widx[pl.ds(16 * m, 16)] = zeros
      return 0
    lax.fori_loop(0, WIN // 16, wz_body, 0)
    for k in range(CHUNK // WIN):
      pltpu.sync_copy(widx, isp.at[pl.ds(base + k * WIN, WIN)])
    @pl.when(sid == 0)
    def _():
      pltpu.sync_copy(widx.at[pl.ds(0, 128)], isp.at[pl.ds(N, 128)])
    pltpu.sync_copy(mask_hbm.at[pl.ds(base, CHUNK)], mch)
    def cs_body(m, carry):
      v = mch[pl.ds(16 * m, 16)]
      posn[pl.ds(16 * m, 16)] = (plsc.cumsum(v) - v) + carry
      return carry + jnp.sum(v)
    tot = lax.fori_loop(0, NV, cs_body, jnp.int32(0))
    t16[...] = zeros + tot
    pltpu.sync_copy(t16, tsp.at[sid])
    plsc.subcore_barrier()
    pltpu.sync_copy(tsp, tall)
    cbase = zeros
    for t in range(NT):
      cbase = cbase + tall[t, :] * jnp.where(jnp.int32(t) < sid, 1, 0)
    def ps_body(m, _):
      v = mch[pl.ds(16 * m, 16)]
      p = posn[pl.ds(16 * m, 16)] + cbase
      dump = jnp.full((16,), N, jnp.int32) + lanes
      pd = jnp.where(v > 0, p, dump)
      posn[pl.ds(16 * m, 16)] = pd
      mch[pl.ds(16 * m, 16)] = base + 16 * m + lanes
      posb[m // 8, pl.ds((m % 8) * 16, 16)] = pd
      return 0
    lax.fori_loop(0, NV, ps_body, 0)
    for j in range(NJ):
      pltpu.async_copy(mch.at[pl.ds(128 * j, 128)], isp.at[posb.at[j]], sem)
    pltpu.make_async_copy(mask_hbm.at[pl.ds(0, CHUNK)], posn, sem).wait()
    plsc.subcore_barrier()
    w = cid * NT + sid
    for win in range(OUT_CHUNK // WIN):
      start = w * OUT_CHUNK + win * WIN
      pltpu.sync_copy(isp.at[pl.ds(start, WIN)], widx)
      for j in range(WIN // 128):
        pltpu.async_copy(
            vert_hbm.at[widx.at[pl.ds(128 * j, 128)]],
            wrows.at[pl.ds(128 * j, 128)], sem)
      pltpu.make_async_copy(vert_hbm.at[pl.ds(0, WIN)], wrows, sem).wait()
      pltpu.sync_copy(wrows, out_hbm.at[pl.ds(start, WIN)])

  return vertex_kernel


_sort_call = _build_sort_kernel()
_vertex_call = _build_vertex_kernel()


def kernel(vertices, objects, mask):
  path_len = vertices.shape[-2]
  obj2d = objects.reshape(-1, objects.shape[-1]).astype(jnp.int32)
  groups = _sort_call(obj2d)
  v2d = vertices.reshape(-1, path_len * 3).astype(jnp.float32)
  if mask is not None:
    m32 = mask.reshape(-1).astype(jnp.int32)
    mv = _vertex_call(m32, v2d)
  else:
    mv = _vertex_call(jnp.ones((v2d.shape[0],), jnp.int32), v2d)
  masked_vertices = mv.reshape(-1, path_len, 3)
  groups = groups.reshape(objects.shape[:-1])
  return masked_vertices, groups


# speedup vs baseline: 1.2307x; 1.0011x over previous
import functools

import jax
import jax.numpy as jnp
from jax import lax
from jax.experimental import pallas as pl
from jax.experimental.pallas import tpu as pltpu
from jax.experimental.pallas import tpu_sc as plsc

N = 65536
PL = 16
NT = 16
CHUNK = N // NT
NV = CHUNK // 16
NJ = CHUNK // 128
NB = 1024
VROW = 48
OUT_CHUNK = N // 32
WIN = 512

_PARAMS = pltpu.CompilerParams(
    needs_layout_passes=False, use_tc_tiling_on_sc=False)


def _iota16():
  return lax.iota(jnp.int32, 16)


def _build_sort_kernel():
  mesh = plsc.VectorSubcoreMesh(
      core_axis_name="c", subcore_axis_name="s", num_cores=1)

  @functools.partial(
      pl.kernel,
      mesh=mesh,
      compiler_params=_PARAMS,
      out_type=jax.ShapeDtypeStruct((N,), jnp.int32),
      scratch_types=[
          pltpu.VMEM((CHUNK,), jnp.int32),
          pltpu.VMEM((CHUNK,), jnp.int32),
          pltpu.VMEM((CHUNK,), jnp.int32),
          pltpu.VMEM((CHUNK,), jnp.int32),
          pltpu.VMEM((NJ, 128), jnp.int32),
          pltpu.VMEM((NB,), jnp.int32),
          pltpu.VMEM((NB,), jnp.int32),
          pltpu.VMEM((NT, NB), jnp.int32),
          pltpu.VMEM((CHUNK,), jnp.int32),
          pltpu.VMEM((CHUNK, PL), jnp.int32),
          pltpu.VMEM((8,), jnp.int32),
          pltpu.VMEM((8, PL), jnp.int32),
          pltpu.VMEM((16,), jnp.int32),
          pltpu.VMEM((NT, 16), jnp.int32),
          pltpu.VMEM_SHARED((2 * N,), jnp.int32),
          pltpu.VMEM_SHARED((2 * N,), jnp.int32),
          pltpu.VMEM_SHARED((NT, NB), jnp.int32),
          pltpu.VMEM_SHARED((NT, 16), jnp.int32),
          pltpu.SemaphoreType.DMA,
      ],
  )
  def sort_kernel(obj2d_hbm, groups_hbm, idxc, fidx, colv, pkdc, posb,
                  hist, offs, hall, flags, rows, pidx, prow, t16, tall,
                  idxsp, pkdsp, hsp, tsp, sem):
    sid = lax.axis_index("s")
    base = sid * CHUNK
    lanes = _iota16()
    ones = jnp.ones((16,), jnp.int32)
    zeros = jnp.zeros((16,), jnp.int32)

    def init_body(m, _):
      idxc[pl.ds(16 * m, 16)] = base + 16 * m + lanes
      return 0
    lax.fori_loop(0, NV, init_body, 0)
    pltpu.sync_copy(idxc, idxsp.at[pl.ds(base, CHUNK)])
    plsc.subcore_barrier()

    def pass_body(p, _):
      col = 15 - p
      r = lax.rem(p, 3)
      src = pl.multiple_of(lax.rem(p, 2) * N + base, 8)
      dst_off = (1 - lax.rem(p, 2)) * N
      with jax.named_scope("rs_gather"):
        pltpu.sync_copy(idxsp.at[pl.ds(src, CHUNK)], idxc)
        @pl.when(r == 0)
        def _():
          for j in range(NJ):
            pltpu.async_copy(
                obj2d_hbm.at[idxc.at[pl.ds(128 * j, 128)]],
                rows.at[pl.ds(128 * j, 128)], sem)
          pltpu.make_async_copy(
              obj2d_hbm.at[pl.ds(0, CHUNK)], rows, sem).wait()
          c0 = zeros + col
          c1 = jnp.maximum(c0 - 1, 0)
          c2 = jnp.maximum(c0 - 2, 0)
          def ex_body(m, _):
            for h in range(4):
              mm = 4 * m + h
              pos = 16 * mm + lanes
              colv[pl.ds(16 * mm, 16)] = plsc.load_gather(rows, [pos, c0])
              d1 = plsc.load_gather(rows, [pos, c1])
              d2 = plsc.load_gather(rows, [pos, c2])
              pkdc[pl.ds(16 * mm, 16)] = d1 * 1024 + d2
            return 0
          lax.fori_loop(0, NV // 4, ex_body, 0)
        @pl.when(r != 0)
        def _():
          pltpu.sync_copy(pkdsp.at[pl.ds(src, CHUNK)], pkdc)
          @pl.when(r == 1)
          def _():
            def s1_body(m, _):
              for h in range(4):
                mm = 4 * m + h
                v = pkdc[pl.ds(16 * mm, 16)]
                colv[pl.ds(16 * mm, 16)] = lax.shift_right_logical(v, 10)
                pkdc[pl.ds(16 * mm, 16)] = jnp.bitwise_and(v, 1023)
              return 0
            lax.fori_loop(0, NV // 4, s1_body, 0)
          @pl.when(r == 2)
          def _():
            def s2_body(m, _):
              for h in range(4):
                mm = 4 * m + h
                colv[pl.ds(16 * mm, 16)] = pkdc[pl.ds(16 * mm, 16)]
              return 0
            lax.fori_loop(0, NV // 4, s2_body, 0)
      with jax.named_scope("rs_hist"):
        def hz_body(m, _):
          hist[pl.ds(16 * m, 16)] = zeros
          return 0
        lax.fori_loop(0, NB // 16, hz_body, 0)
        def dig_body(m, _):
          for h in range(4):
            mm = 4 * m + h
            plsc.addupdate_scatter(hist, [colv[pl.ds(16 * mm, 16)]], ones)
          return 0
        lax.fori_loop(0, NV // 4, dig_body, 0)
      with jax.named_scope("rs_xchg"):
        pltpu.sync_copy(hist, hsp.at[sid])
        plsc.subcore_barrier()
        pltpu.sync_copy(hsp, hall)
      with jax.named_scope("rs_scan"):
        def scan_body(k, carry):
          tot = zeros
          part = zeros
          for t in range(NT):
            h = hall[t, pl.ds(16 * k, 16)]
            tot = tot + h
            part = part + h * jnp.where(jnp.int32(t) < sid, 1, 0)
          incl = plsc.cumsum(tot)
          offs[pl.ds(16 * k, 16)] = carry + (incl - tot) + part
          return carry + jnp.sum(tot)
        lax.fori_loop(0, NB // 16, scan_body, jnp.int32(0))
      with jax.named_scope("rs_perm"):
        dvec = zeros + dst_off
        def perm_body(m, _):
          for h in range(2):
            mm = 2 * m + h
            d = colv[pl.ds(16 * mm, 16)]
            b = plsc.load_gather(offs, [d])
            cnt, _ = plsc.scan_count(d)
            posb[mm // 8, pl.ds((mm % 8) * 16, 16)] = b + cnt - 1 + dvec
            plsc.addupdate_scatter(offs, [d], ones)
          return 0
        lax.fori_loop(0, NV // 2, perm_body, 0)
      with jax.named_scope("rs_scat"):
        for j in range(NJ):
          pltpu.async_copy(
              idxc.at[pl.ds(128 * j, 128)], idxsp.at[posb.at[j]], sem)
        pltpu.make_async_copy(
            groups_hbm.at[pl.ds(0, CHUNK)], fidx, sem).wait()
        @pl.when(jnp.logical_and(r != 2, p != 15))
        def _():
          for j in range(NJ):
            pltpu.async_copy(
                pkdc.at[pl.ds(128 * j, 128)], pkdsp.at[posb.at[j]], sem)
          pltpu.make_async_copy(
              groups_hbm.at[pl.ds(0, CHUNK)], fidx, sem).wait()
        plsc.subcore_barrier()
      return 0
    lax.fori_loop(0, 16, pass_body, 0)

    pltpu.sync_copy(idxsp.at[pl.ds(base, CHUNK)], idxc)
    for j in range(NJ):
      pltpu.async_copy(
          obj2d_hbm.at[idxc.at[pl.ds(128 * j, 128)]],
          rows.at[pl.ds(128 * j, 128)], sem)
    pltpu.make_async_copy(obj2d_hbm.at[pl.ds(0, CHUNK)], rows, sem).wait()
    pb = pl.multiple_of(jnp.maximum(base - 8, 0), 8)
    pltpu.sync_copy(idxsp.at[pl.ds(pb, 8)], pidx)
    pltpu.async_copy(obj2d_hbm.at[pidx], prow, sem).wait()

    def cmp_body(m, _):
      p = 16 * m + lanes
      pp = jnp.maximum(p - 1, 0)
      acc = zeros
      for jcol in range(PL):
        cj = jnp.full((16,), jcol, jnp.int32)
        cur = plsc.load_gather(rows, [p, cj])
        prv = plsc.load_gather(rows, [pp, cj])
        acc = acc | jnp.where(cur != prv, 1, 0)
      flags[pl.ds(16 * m, 16)] = acc
      return 0
    lax.fori_loop(0, NV, cmp_body, 0)
    first = rows[0, :]
    prev = prow[7, :]
    df = jnp.sum(jnp.where(first != prev, 1, 0))
    f0 = jnp.where(sid == 0, jnp.int32(0), jnp.minimum(df, 1))
    v0 = flags[pl.ds(0, 16)]
    flags[pl.ds(0, 16)] = jnp.where(lanes == 0, f0, v0)

    def sum_body(m, carry):
      f = flags[pl.ds(16 * m, 16)]
      colv[pl.ds(16 * m, 16)] = plsc.cumsum(f) + carry
      return carry + jnp.sum(f)
    tot = lax.fori_loop(0, NV, sum_body, jnp.int32(0))
    t16[...] = zeros + tot
    pltpu.sync_copy(t16, tsp.at[sid])
    plsc.subcore_barrier()
    pltpu.sync_copy(tsp, tall)
    rbase = zeros
    for t in range(NT):
      rbase = rbase + tall[t, :] * jnp.where(jnp.int32(t) < sid, 1, 0)
    def add_body(m, _):
      colv[pl.ds(16 * m, 16)] = colv[pl.ds(16 * m, 16)] + rbase
      posb[m // 8, pl.ds((m % 8) * 16, 16)] = idxc[pl.ds(16 * m, 16)]
      return 0
    lax.fori_loop(0, NV, add_body, 0)
    for j in range(NJ):
      pltpu.async_copy(
          colv.at[pl.ds(128 * j, 128)], groups_hbm.at[posb.at[j]], sem)
    pltpu.make_async_copy(groups_hbm.at[pl.ds(0, CHUNK)], fidx, sem).wait()

  return sort_kernel


def _build_vertex_kernel():
  mesh = plsc.VectorSubcoreMesh(
      core_axis_name="c", subcore_axis_name="s", num_cores=2)

  @functools.partial(
      pl.kernel,
      mesh=mesh,
      compiler_params=_PARAMS,
      out_type=jax.ShapeDtypeStruct((N, VROW), jnp.float32),
      scratch_types=[
          pltpu.VMEM((CHUNK,), jnp.int32),
          pltpu.VMEM((CHUNK,), jnp.int32),
          pltpu.VMEM((NJ, 128), jnp.int32),
          pltpu.VMEM((WIN,), jnp.int32),
          pltpu.VMEM((WIN, VROW), jnp.float32),
          pltpu.VMEM((16,), jnp.int32),
          pltpu.VMEM((NT, 16), jnp.int32),
          pltpu.VMEM_SHARED((N + 128,), jnp.int32),
          pltpu.VMEM_SHARED((NT, 16), jnp.int32),
          pltpu.SemaphoreType.DMA,
      ],
  )
  def vertex_kernel(mask_hbm, vert_hbm, out_hbm, mch, posn, posb, widx,
                    wrows, t16, tall, isp, tsp, sem):
    cid = lax.axis_index("c")
    sid = lax.axis_index("s")
    base = sid * CHUNK
    lanes = _iota16()
    zeros = jnp.zeros((16,), jnp.int32)

    def wz_body(m, _):
      widx[pl.ds(16 * m, 16)] = zeros
      return 0
    lax.fori_loop(0, WIN // 16, wz_body, 0)
    for k in range(CHUNK // WIN):
      pltpu.sync_copy(widx, isp.at[pl.ds(base + k * WIN, WIN)])
    @pl.when(sid == 0)
    def _():
      pltpu.sync_copy(widx.at[pl.ds(0, 128)], isp.at[pl.ds(N, 128)])
    pltpu.sync_copy(mask_hbm.at[pl.ds(base, CHUNK)], mch)
    def cs_body(m, carry):
      v = mch[pl.ds(16 * m, 16)]
      posn[pl.ds(16 * m, 16)] = (plsc.cumsum(v) - v) + carry
      return carry + jnp.sum(v)
    tot = lax.fori_loop(0, NV, cs_body, jnp.int32(0))
    t16[...] = zeros + tot
    pltpu.sync_copy(t16, tsp.at[sid])
    plsc.subcore_barrier()
    pltpu.sync_copy(tsp, tall)
    cbase = zeros
    for t in range(NT):
      cbase = cbase + tall[t, :] * jnp.where(jnp.int32(t) < sid, 1, 0)
    def ps_body(m, _):
      v = mch[pl.ds(16 * m, 16)]
      p = posn[pl.ds(16 * m, 16)] + cbase
      dump = jnp.full((16,), N, jnp.int32) + lanes
      pd = jnp.where(v > 0, p, dump)
      posn[pl.ds(16 * m, 16)] = pd
      mch[pl.ds(16 * m, 16)] = base + 16 * m + lanes
      posb[m // 8, pl.ds((m % 8) * 16, 16)] = pd
      return 0
    lax.fori_loop(0, NV, ps_body, 0)
    for j in range(NJ):
      pltpu.async_copy(mch.at[pl.ds(128 * j, 128)], isp.at[posb.at[j]], sem)
    pltpu.make_async_copy(mask_hbm.at[pl.ds(0, CHUNK)], posn, sem).wait()
    plsc.subcore_barrier()
    w = cid * NT + sid
    for win in range(OUT_CHUNK // WIN):
      start = w * OUT_CHUNK + win * WIN
      pltpu.sync_copy(isp.at[pl.ds(start, WIN)], widx)
      for j in range(WIN // 128):
        pltpu.async_copy(
            vert_hbm.at[widx.at[pl.ds(128 * j, 128)]],
            wrows.at[pl.ds(128 * j, 128)], sem)
      pltpu.make_async_copy(vert_hbm.at[pl.ds(0, WIN)], wrows, sem).wait()
      pltpu.sync_copy(wrows, out_hbm.at[pl.ds(start, WIN)])

  return vertex_kernel


_sort_call = _build_sort_kernel()
_vertex_call = _build_vertex_kernel()


def kernel(vertices, objects, mask):
  path_len = vertices.shape[-2]
  obj2d = objects.reshape(-1, objects.shape[-1]).astype(jnp.int32)
  groups = _sort_call(obj2d)
  v2d = vertices.reshape(-1, path_len * 3).astype(jnp.float32)
  if mask is not None:
    m32 = mask.reshape(-1).astype(jnp.int32)
  else:
    m32 = jnp.ones((v2d.shape[0],), jnp.int32)
  mv = _vertex_call(m32, v2d)
  masked_vertices = mv.reshape(-1, path_len, 3)
  groups = groups.reshape(objects.shape[:-1])
  return masked_vertices, groups


# speedup vs baseline: 1.2320x; 1.0011x over previous
import functools

import jax
import jax.numpy as jnp
from jax import lax
from jax.experimental import pallas as pl
from jax.experimental.pallas import tpu as pltpu
from jax.experimental.pallas import tpu_sc as plsc

N = 65536
PL = 16
NT = 16
CHUNK = N // NT
NV = CHUNK // 16
NJ = CHUNK // 128
NB = 1024
VROW = 48
OUT_CHUNK = N // 32
WIN = 512

_PARAMS = pltpu.CompilerParams(
    needs_layout_passes=False, use_tc_tiling_on_sc=False)


def _iota16():
  return lax.iota(jnp.int32, 16)


def _build_sort_kernel():
  mesh = plsc.VectorSubcoreMesh(
      core_axis_name="c", subcore_axis_name="s", num_cores=1)

  @functools.partial(
      pl.kernel,
      mesh=mesh,
      compiler_params=_PARAMS,
      out_type=jax.ShapeDtypeStruct((N,), jnp.int32),
      scratch_types=[
          pltpu.VMEM((CHUNK,), jnp.int32),
          pltpu.VMEM((CHUNK,), jnp.int32),
          pltpu.VMEM((CHUNK,), jnp.int32),
          pltpu.VMEM((CHUNK,), jnp.int32),
          pltpu.VMEM((NJ, 128), jnp.int32),
          pltpu.VMEM((NB,), jnp.int32),
          pltpu.VMEM((NB,), jnp.int32),
          pltpu.VMEM((NT, NB), jnp.int32),
          pltpu.VMEM((CHUNK,), jnp.int32),
          pltpu.VMEM((CHUNK, PL), jnp.int32),
          pltpu.VMEM((8,), jnp.int32),
          pltpu.VMEM((8, PL), jnp.int32),
          pltpu.VMEM((16,), jnp.int32),
          pltpu.VMEM((NT, 16), jnp.int32),
          pltpu.VMEM_SHARED((2 * N,), jnp.int32),
          pltpu.VMEM_SHARED((2 * N,), jnp.int32),
          pltpu.VMEM_SHARED((NT, NB), jnp.int32),
          pltpu.VMEM_SHARED((NT, 16), jnp.int32),
          pltpu.SemaphoreType.DMA,
      ],
  )
  def sort_kernel(obj2d_hbm, groups_hbm, idxc, fidx, colv, pkdc, posb,
                  hist, offs, hall, flags, rows, pidx, prow, t16, tall,
                  idxsp, pkdsp, hsp, tsp, sem):
    sid = lax.axis_index("s")
    base = sid * CHUNK
    lanes = _iota16()
    ones = jnp.ones((16,), jnp.int32)
    zeros = jnp.zeros((16,), jnp.int32)

    def init_body(m, _):
      idxc[pl.ds(16 * m, 16)] = base + 16 * m + lanes
      return 0
    lax.fori_loop(0, NV, init_body, 0)
    pltpu.sync_copy(idxc, idxsp.at[pl.ds(base, CHUNK)])
    plsc.subcore_barrier()

    def pass_body(p, _):
      col = 15 - p
      r = lax.rem(p, 3)
      src = pl.multiple_of(lax.rem(p, 2) * N + base, 8)
      dst_off = (1 - lax.rem(p, 2)) * N
      with jax.named_scope("rs_gather"):
        pltpu.sync_copy(idxsp.at[pl.ds(src, CHUNK)], idxc)
        @pl.when(r == 0)
        def _():
          for j in range(NJ):
            pltpu.async_copy(
                obj2d_hbm.at[idxc.at[pl.ds(128 * j, 128)]],
                rows.at[pl.ds(128 * j, 128)], sem)
          pltpu.make_async_copy(
              obj2d_hbm.at[pl.ds(0, CHUNK)], rows, sem).wait()
          c0 = zeros + col
          c1 = jnp.maximum(c0 - 1, 0)
          c2 = jnp.maximum(c0 - 2, 0)
          def ex_body(m, _):
            for h in range(4):
              mm = 4 * m + h
              pos = 16 * mm + lanes
              colv[pl.ds(16 * mm, 16)] = plsc.load_gather(rows, [pos, c0])
              d1 = plsc.load_gather(rows, [pos, c1])
              d2 = plsc.load_gather(rows, [pos, c2])
              pkdc[pl.ds(16 * mm, 16)] = d1 * 1024 + d2
            return 0
          lax.fori_loop(0, NV // 4, ex_body, 0)
        @pl.when(r != 0)
        def _():
          pltpu.sync_copy(pkdsp.at[pl.ds(src, CHUNK)], pkdc)
          @pl.when(r == 1)
          def _():
            def s1_body(m, _):
              for h in range(4):
                mm = 4 * m + h
                v = pkdc[pl.ds(16 * mm, 16)]
                colv[pl.ds(16 * mm, 16)] = lax.shift_right_logical(v, 10)
                pkdc[pl.ds(16 * mm, 16)] = jnp.bitwise_and(v, 1023)
              return 0
            lax.fori_loop(0, NV // 4, s1_body, 0)
          @pl.when(r == 2)
          def _():
            def s2_body(m, _):
              for h in range(4):
                mm = 4 * m + h
                colv[pl.ds(16 * mm, 16)] = pkdc[pl.ds(16 * mm, 16)]
              return 0
            lax.fori_loop(0, NV // 4, s2_body, 0)
      with jax.named_scope("rs_hist"):
        def hz_body(m, _):
          hist[pl.ds(16 * m, 16)] = zeros
          return 0
        lax.fori_loop(0, NB // 16, hz_body, 0)
        def dig_body(m, _):
          for h in range(4):
            mm = 4 * m + h
            plsc.addupdate_scatter(hist, [colv[pl.ds(16 * mm, 16)]], ones)
          return 0
        lax.fori_loop(0, NV // 4, dig_body, 0)
      with jax.named_scope("rs_xchg"):
        pltpu.sync_copy(hist, hsp.at[sid])
        plsc.subcore_barrier()
        pltpu.sync_copy(hsp, hall)
      with jax.named_scope("rs_scan"):
        def scan_body(k, carry):
          tot = zeros
          part = zeros
          for t in range(NT):
            h = hall[t, pl.ds(16 * k, 16)]
            tot = tot + h
            part = part + h * jnp.where(jnp.int32(t) < sid, 1, 0)
          incl = plsc.cumsum(tot)
          offs[pl.ds(16 * k, 16)] = carry + (incl - tot) + part
          return carry + jnp.sum(tot)
        lax.fori_loop(0, NB // 16, scan_body, jnp.int32(0))
      with jax.named_scope("rs_perm"):
        dvec = zeros + dst_off
        def perm_body(m, _):
          for h in range(2):
            mm = 2 * m + h
            d = colv[pl.ds(16 * mm, 16)]
            b = plsc.load_gather(offs, [d])
            cnt, _ = plsc.scan_count(d)
            posb[mm // 8, pl.ds((mm % 8) * 16, 16)] = b + cnt - 1 + dvec
            plsc.addupdate_scatter(offs, [d], ones)
          return 0
        lax.fori_loop(0, NV // 2, perm_body, 0)
      with jax.named_scope("rs_scat"):
        for j in range(NJ):
          pltpu.async_copy(
              idxc.at[pl.ds(128 * j, 128)], idxsp.at[posb.at[j]], sem)
        pltpu.make_async_copy(
            groups_hbm.at[pl.ds(0, CHUNK)], fidx, sem).wait()
        @pl.when(jnp.logical_and(r != 2, p != 15))
        def _():
          for j in range(NJ):
            pltpu.async_copy(
                pkdc.at[pl.ds(128 * j, 128)], pkdsp.at[posb.at[j]], sem)
          pltpu.make_async_copy(
              groups_hbm.at[pl.ds(0, CHUNK)], fidx, sem).wait()
        plsc.subcore_barrier()
      return 0
    lax.fori_loop(0, 16, pass_body, 0)

    pltpu.sync_copy(idxsp.at[pl.ds(base, CHUNK)], idxc)
    for j in range(NJ):
      pltpu.async_copy(
          obj2d_hbm.at[idxc.at[pl.ds(128 * j, 128)]],
          rows.at[pl.ds(128 * j, 128)], sem)
    pltpu.make_async_copy(obj2d_hbm.at[pl.ds(0, CHUNK)], rows, sem).wait()
    pb = pl.multiple_of(jnp.maximum(base - 8, 0), 8)
    pltpu.sync_copy(idxsp.at[pl.ds(pb, 8)], pidx)
    pltpu.async_copy(obj2d_hbm.at[pidx], prow, sem).wait()

    def cmp_body(m, _):
      p = 16 * m + lanes
      pp = jnp.maximum(p - 1, 0)
      acc = zeros
      for jcol in range(PL):
        cj = jnp.full((16,), jcol, jnp.int32)
        cur = plsc.load_gather(rows, [p, cj])
        prv = plsc.load_gather(rows, [pp, cj])
        acc = acc | jnp.where(cur != prv, 1, 0)
      flags[pl.ds(16 * m, 16)] = acc
      return 0
    lax.fori_loop(0, NV, cmp_body, 0)
    first = rows[0, :]
    prev = prow[7, :]
    df = jnp.sum(jnp.where(first != prev, 1, 0))
    f0 = jnp.where(sid == 0, jnp.int32(0), jnp.minimum(df, 1))
    v0 = flags[pl.ds(0, 16)]
    flags[pl.ds(0, 16)] = jnp.where(lanes == 0, f0, v0)

    def sum_body(m, carry):
      f = flags[pl.ds(16 * m, 16)]
      colv[pl.ds(16 * m, 16)] = plsc.cumsum(f) + carry
      return carry + jnp.sum(f)
    tot = lax.fori_loop(0, NV, sum_body, jnp.int32(0))
    t16[...] = zeros + tot
    pltpu.sync_copy(t16, tsp.at[sid])
    plsc.subcore_barrier()
    pltpu.sync_copy(tsp, tall)
    rbase = zeros
    for t in range(NT):
      rbase = rbase + tall[t, :] * jnp.where(jnp.int32(t) < sid, 1, 0)
    def add_body(m, _):
      colv[pl.ds(16 * m, 16)] = colv[pl.ds(16 * m, 16)] + rbase
      posb[m // 8, pl.ds((m % 8) * 16, 16)] = idxc[pl.ds(16 * m, 16)]
      return 0
    lax.fori_loop(0, NV, add_body, 0)
    for j in range(NJ):
      pltpu.async_copy(
          colv.at[pl.ds(128 * j, 128)], groups_hbm.at[posb.at[j]], sem)
    pltpu.make_async_copy(groups_hbm.at[pl.ds(0, CHUNK)], fidx, sem).wait()

  return sort_kernel


def _build_vertex_kernel():
  mesh = plsc.VectorSubcoreMesh(
      core_axis_name="c", subcore_axis_name="s", num_cores=2)

  @functools.partial(
      pl.kernel,
      mesh=mesh,
      compiler_params=_PARAMS,
      out_type=jax.ShapeDtypeStruct((N, VROW), jnp.float32),
      scratch_types=[
          pltpu.VMEM((CHUNK,), jnp.int32),
          pltpu.VMEM((CHUNK,), jnp.int32),
          pltpu.VMEM((NJ, 128), jnp.int32),
          pltpu.VMEM((WIN,), jnp.int32),
          pltpu.VMEM((WIN, VROW), jnp.float32),
          pltpu.VMEM((16,), jnp.int32),
          pltpu.VMEM((NT, 16), jnp.int32),
          pltpu.VMEM_SHARED((N + 128,), jnp.int32),
          pltpu.VMEM_SHARED((NT, 16), jnp.int32),
          pltpu.SemaphoreType.DMA,
      ],
  )
  def vertex_kernel(mask_hbm, vert_hbm, out_hbm, mch, posn, posb, widx,
                    wrows, t16, tall, isp, tsp, sem):
    cid = lax.axis_index("c")
    sid = lax.axis_index("s")
    base = sid * CHUNK
    lanes = _iota16()
    zeros = jnp.zeros((16,), jnp.int32)

    def wz_body(m, _):
      widx[pl.ds(16 * m, 16)] = zeros
      return 0
    lax.fori_loop(0, WIN // 16, wz_body, 0)
    for k in range(CHUNK // WIN):
      pltpu.sync_copy(widx, isp.at[pl.ds(base + k * WIN, WIN)])
    @pl.when(sid == 0)
    def _():
      pltpu.sync_copy(widx.at[pl.ds(0, 128)], isp.at[pl.ds(N, 128)])
    pltpu.sync_copy(mask_hbm.at[pl.ds(base, CHUNK)], mch)
    def cs_body(m, carry):
      v = mch[pl.ds(16 * m, 16)]
      posn[pl.ds(16 * m, 16)] = (plsc.cumsum(v) - v) + carry
      return carry + jnp.sum(v)
    tot = lax.fori_loop(0, NV, cs_body, jnp.int32(0))
    t16[...] = zeros + tot
    pltpu.sync_copy(t16, tsp.at[sid])
    plsc.subcore_barrier()
    pltpu.sync_copy(tsp, tall)
    cbase = zeros
    for t in range(NT):
      cbase = cbase + tall[t, :] * jnp.where(jnp.int32(t) < sid, 1, 0)
    def ps_body(m, _):
      v = mch[pl.ds(16 * m, 16)]
      p = posn[pl.ds(16 * m, 16)] + cbase
      dump = jnp.full((16,), N, jnp.int32) + lanes
      pd = jnp.where(v > 0, p, dump)
      posn[pl.ds(16 * m, 16)] = pd
      mch[pl.ds(16 * m, 16)] = base + 16 * m + lanes
      posb[m // 8, pl.ds((m % 8) * 16, 16)] = pd
      return 0
    lax.fori_loop(0, NV, ps_body, 0)
    for j in range(NJ):
      pltpu.async_copy(mch.at[pl.ds(128 * j, 128)], isp.at[posb.at[j]], sem)
    pltpu.make_async_copy(mask_hbm.at[pl.ds(0, CHUNK)], posn, sem).wait()
    plsc.subcore_barrier()
    w = cid * NT + sid
    for win in range(OUT_CHUNK // WIN):
      start = w * OUT_CHUNK + win * WIN
      pltpu.sync_copy(isp.at[pl.ds(start, WIN)], widx)
      for j in range(WIN // 128):
        pltpu.async_copy(
            vert_hbm.at[widx.at[pl.ds(128 * j, 128)]],
            wrows.at[pl.ds(128 * j, 128)], sem)
      pltpu.make_async_copy(vert_hbm.at[pl.ds(0, WIN)], wrows, sem).wait()
      pltpu.sync_copy(wrows, out_hbm.at[pl.ds(start, WIN)])

  return vertex_kernel


_CALLS = {}


def _get_calls():
  if "sort" not in _CALLS:
    _CALLS["sort"] = _build_sort_kernel()
    _CALLS["vertex"] = _build_vertex_kernel()
  return _CALLS["sort"], _CALLS["vertex"]


def kernel(vertices, objects, mask):
  _sort_call, _vertex_call = _get_calls()
  path_len = vertices.shape[-2]
  obj2d = objects.reshape(-1, objects.shape[-1]).astype(jnp.int32)
  groups = _sort_call(obj2d)
  v2d = vertices.reshape(-1, path_len * 3).astype(jnp.float32)
  if mask is not None:
    m32 = mask.reshape(-1).astype(jnp.int32)
  else:
    m32 = jnp.ones((v2d.shape[0],), jnp.int32)
  mv = _vertex_call(m32, v2d)
  masked_vertices = mv.reshape(-1, path_len, 3)
  groups = groups.reshape(objects.shape[:-1])
  return masked_vertices, groups
